# baseline (speedup 1.0000x reference)
import functools

import jax
import jax.numpy as jnp
from jax import lax
from jax.experimental import pallas as pl
from jax.experimental.pallas import tpu as pltpu
from jax.experimental.pallas import tpu_sc as plsc

_VOCAB = 1000000
_D = 64
_MAX_LEN = 200
_BATCH = 4096
_B = _BATCH * _MAX_LEN

_NC = 2
_NS = 16
_NW = _NC * _NS
_PER_W = _B // _NW
_C = 200
_NCHUNK = _PER_W // _C
_L = 16
_NPAIR = _C // 2
_PER_W2 = _PER_W // 2
_GOFF = [16 * q for q in range(12)] + [184]


def _positional() -> jax.Array:
    position = jnp.arange(0, _MAX_LEN, dtype=jnp.float32).reshape(-1, 1)
    div = jnp.exp(
        jnp.arange(0, _D, 2, dtype=jnp.float32) / _D
        * -jnp.log(jnp.float32(10000.0))
    )
    p = jnp.zeros((_MAX_LEN, _D), dtype=jnp.float32)
    p = p.at[:, 0::2].set(jnp.sin(position * div))
    p = p.at[:, 1::2].set(jnp.cos(position * div))
    return p


def _splat_lane(v, lane):
    idx = jnp.full((_L, 1), lane, dtype=jnp.int32)
    return lax.gather(
        v, idx,
        lax.GatherDimensionNumbers(
            offset_dims=(), collapsed_slice_dims=(0,), start_index_map=(0,)),
        slice_sizes=(1,),
        mode=lax.GatherScatterMode.PROMISE_IN_BOUNDS)


_mesh = plsc.VectorSubcoreMesh(core_axis_name="c", subcore_axis_name="s")


@functools.partial(
    pl.kernel,
    mesh=_mesh,
    out_type=jax.ShapeDtypeStruct((_B // 2, 128), jnp.float32),
    scratch_types=[
        pltpu.VMEM((4 * _C,), jnp.int32),
        pltpu.VMEM((2, _C, 128), jnp.float32),
        pltpu.VMEM((2, 2 * _NPAIR, 128), jnp.float32),
        pltpu.VMEM((_NPAIR, 128), jnp.float32),
        pltpu.SemaphoreType.DMA((4,)),
        pltpu.SemaphoreType.DMA((2,)),
        pltpu.SemaphoreType.DMA((2,)),
    ],
    compiler_params=pltpu.CompilerParams(needs_layout_passes=False),
)
def _embed(x_hbm, table_hbm, p_hbm, out_hbm,
           idxb, rows, outb, p_v, isem, gsem, ssem):
    wid = lax.axis_index("s") * _NC + lax.axis_index("c")
    base = wid * _PER_W
    obase = wid * _PER_W2
    pltpu.sync_copy(p_hbm, p_v)

    def idx_load(k, slot):
        pltpu.async_copy(
            x_hbm.at[pl.ds(pl.multiple_of(base + k * _C, 8), _C)],
            idxb.at[pl.ds(pl.multiple_of(slot * _C, 8), _C)],
            isem.at[slot])

    def idx_wait(k, slot):
        pltpu.make_async_copy(
            x_hbm.at[pl.ds(pl.multiple_of(base + k * _C, 8), _C)],
            idxb.at[pl.ds(pl.multiple_of(slot * _C, 8), _C)],
            isem.at[slot]).wait()

    def gathers(slot, rb):
        for off in _GOFF:
            idxv = idxb[pl.ds(pl.multiple_of(slot * _C + off, 8), _L)]
            pltpu.async_copy(
                table_hbm.at[(idxv >> 1)],
                rows.at[rb].at[pl.ds(off, _L)],
                gsem.at[rb],
            )

    def gathers_wait(slot, rb):
        for off in _GOFF:
            idxv = idxb[pl.ds(pl.multiple_of(slot * _C + off, 8), _L)]
            pltpu.make_async_copy(
                table_hbm.at[(idxv >> 1)],
                rows.at[rb].at[pl.ds(off, _L)],
                gsem.at[rb],
            ).wait()

    def store(k, os):
        pltpu.async_copy(
            outb.at[os],
            out_hbm.at[pl.ds(
                pl.multiple_of(obase + (k - 1) * _NPAIR, 8), 2 * _NPAIR)],
            ssem.at[os],
        )

    def store_wait(k, os):
        pltpu.make_async_copy(
            outb.at[os],
            out_hbm.at[pl.ds(
                pl.multiple_of(obase + (k - 1) * _NPAIR, 8), 2 * _NPAIR)],
            ssem.at[os],
        ).wait()

    def do_pair(rb, os, selv, j_lane0, p_row, o_row):
        lane = lax.iota(jnp.int32, _L)
        row0 = jnp.full((_L,), 2 * p_row, dtype=jnp.int32)
        row1 = row0 + 1
        s0 = _splat_lane(selv, j_lane0)
        s1 = _splat_lane(selv, j_lane0 + 1)
        for d in range(4):
            col = s0 + lane + d * _L
            v = plsc.load_gather(rows.at[rb], [row0, col])
            sl = pl.ds(d * _L, _L)
            outb[os, o_row, sl] = v + p_v[p_row, sl]
        for d in range(4):
            col = s1 + lane + d * _L
            v = plsc.load_gather(rows.at[rb], [row1, col])
            sl = pl.ds(64 + d * _L, _L)
            outb[os, o_row, sl] = v + p_v[p_row, pl.ds(64 + d * _L, _L)]

    def compute(slot, rb, os, ho):
        def group_body(g, carry):
            idxv = idxb[pl.ds(pl.multiple_of(slot * _C + g * _L, 8), _L)]
            selv = (idxv & 1) << 6
            for j in range(8):
                p_row = g * 8 + j
                do_pair(rb, os, selv, 2 * j, p_row, ho + p_row)
            return carry

        lax.fori_loop(0, 12, group_body, 0)
        idxv = idxb[pl.ds(pl.multiple_of(slot * _C + 184, 8), _L)]
        selv = (idxv & 1) << 6
        for j in range(4):
            p_row = 96 + j
            do_pair(rb, os, selv, 8 + 2 * j, p_row, ho + p_row)

    pltpu.sync_copy(x_hbm.at[pl.ds(pl.multiple_of(base, 8), _C)],
                    idxb.at[pl.ds(0, _C)])
    idx_load(1, 1)
    gathers(0, 0)

    def chunk_body(k, carry):
        slot = lax.rem(k, 4)
        nslot = lax.rem(k + 1, 4)
        fslot = lax.rem(k + 2, 4)
        rb = lax.rem(k, 2)
        nrb = 1 - rb
        os = lax.rem(k // 2, 2)
        ho = lax.rem(k, 2) * _NPAIR

        @pl.when(k + 1 < _NCHUNK)
        def _fire_next_gathers():
            idx_wait(k + 1, nslot)
            gathers(nslot, nrb)

        @pl.when(k + 2 < _NCHUNK)
        def _fire_next_idx():
            idx_load(k + 2, fslot)

        @pl.when(jnp.logical_and(lax.rem(k, 2) == 0, k >= 4))
        def _drain_old_store():
            store_wait(k - 3, os)

        gathers_wait(slot, rb)
        compute(slot, rb, os, ho)

        @pl.when(lax.rem(k, 2) == 1)
        def _fire_store():
            store(k, os)

        return carry

    lax.fori_loop(0, _NCHUNK, chunk_body, 0)
    store_wait(_NCHUNK - 3, 0)
    store_wait(_NCHUNK - 1, 1)


def kernel(X, table):
    p2 = _positional().reshape(_NPAIR, 128)
    xf = X.reshape(-1)
    table2 = table.reshape(_VOCAB // 2, 128)
    out2 = _embed(xf, table2, p2)
    return out2.reshape(_BATCH, _MAX_LEN, _D)
